# baseline (speedup 1.0000x reference)
import functools

import jax
import jax.numpy as jnp
from jax import lax
from jax.experimental import pallas as pl
from jax.experimental.pallas import tpu as pltpu
from jax.experimental.pallas import tpu_sc as plsc

LENGTH = 200
EMB = 64
BATCH = 4096

NUM_CORES = 2
NUM_SUBCORES = 16
NW = NUM_CORES * NUM_SUBCORES
ROWS_PER_W = BATCH // NW
ROW_F32 = LENGTH * EMB
NBUF = 4
XCHUNK = 64
XWORDS = XCHUNK * LENGTH
TAB_F32 = (LENGTH + 1) * EMB
NFULL = LENGTH // 16
TAIL = LENGTH - NFULL * 16


@functools.partial(
    pl.kernel,
    mesh=plsc.VectorSubcoreMesh(core_axis_name="c", subcore_axis_name="s"),
    out_type=jax.ShapeDtypeStruct((BATCH * LENGTH * EMB,), jnp.float32),
    compiler_params=pltpu.CompilerParams(
        needs_layout_passes=False, use_tc_tiling_on_sc=False),
    scratch_types=[
        pltpu.VMEM((XWORDS + 16,), jnp.int32),
        pltpu.VMEM((TAB_F32,), jnp.float32),
        pltpu.VMEM((NBUF, ROW_F32), jnp.float32),
        [pltpu.SemaphoreType.DMA] * NBUF,
    ],
)
def _pos_embed_sc(x_hbm, table_hbm, out_hbm, xv, tab, buf, wsem):
    wid = lax.axis_index("s") * NUM_CORES + lax.axis_index("c")
    tok_base = wid * ROWS_PER_W * LENGTH
    out_base = wid * ROWS_PER_W * ROW_F32

    pltpu.sync_copy(table_hbm, tab)

    lane = lax.iota(jnp.int32, 16)
    lane64 = lane * EMB
    ones = jnp.ones((16,), jnp.int32)
    zeros = jnp.zeros((16,), jnp.int32)
    tail_lanes = lane < TAIL

    def stage_x(r):
        pltpu.sync_copy(
            x_hbm.at[pl.ds(tok_base + (r >> 6) * XWORDS, XWORDS)],
            xv.at[pl.ds(0, XWORDS)])

    def gather_16(dst, base_src, base_dst, mask=None):
        for c in range(EMB):
            w = plsc.load_gather(tab, [base_src + c])
            if mask is None:
                plsc.store_scatter(dst, [base_dst + c], w)
            else:
                plsc.store_scatter(dst, [base_dst + c], w, mask=mask)

    def copy_row(r, b):
        lr = r & (XCHUNK - 1)
        dst = buf.at[b]

        def step(i, carry):
            v = xv[pl.ds(lr * LENGTH + 16 * i, 16)]
            m = jnp.where(v != 0, ones, zeros)
            s = plsc.cumsum(m)
            pos = (s + carry) * m
            gather_16(dst, pos * EMB, i * (16 * EMB) + lane64)
            return carry + jnp.sum(m)

        carry = lax.fori_loop(0, NFULL, step, jnp.int32(0))

        v = xv[pl.ds(lr * LENGTH + 16 * NFULL, 16)]
        m = jnp.where(tail_lanes & (v != 0), ones, zeros)
        s = plsc.cumsum(m)
        pos = (s + carry) * m
        gather_16(dst, pos * EMB, NFULL * (16 * EMB) + lane64,
                  mask=tail_lanes)

    def start_write(r, b):
        pltpu.async_copy(buf.at[b],
                         out_hbm.at[pl.ds(out_base + r * ROW_F32, ROW_F32)],
                         wsem[b])

    def wait_write(r, b):
        pltpu.make_async_copy(
            buf.at[b],
            out_hbm.at[pl.ds(out_base + r * ROW_F32, ROW_F32)],
            wsem[b]).wait()

    def body(p, carry):
        for b in range(NBUF):
            r = p * NBUF + b

            @pl.when((r & (XCHUNK - 1)) == 0)
            def _(r=r):
                stage_x(r)

            @pl.when(r >= NBUF)
            def _(r=r, b=b):
                wait_write(r - NBUF, b)

            copy_row(r, b)
            start_write(r, b)

        return carry

    lax.fori_loop(0, ROWS_PER_W // NBUF, body, jnp.int32(0))

    for b in range(NBUF):
        wait_write(ROWS_PER_W - NBUF + b, b)


def kernel(X, pos_table):
    out = _pos_embed_sc(X.reshape(BATCH * LENGTH),
                        pos_table.reshape(TAB_F32))
    return out.reshape(BATCH, LENGTH, EMB)


# speedup vs baseline: 3.4549x; 3.4549x over previous
import functools

import jax
import jax.numpy as jnp
from jax import lax
from jax.experimental import pallas as pl
from jax.experimental.pallas import tpu as pltpu
from jax.experimental.pallas import tpu_sc as plsc

LENGTH = 200
EMB = 64
BATCH = 4096

NUM_CORES = 2
NUM_SUBCORES = 16
NW = NUM_CORES * NUM_SUBCORES
ROWS_PER_W = BATCH // NW
ROW_F32 = LENGTH * EMB
NBUF = 4
XCHUNK = 64
XWORDS = XCHUNK * LENGTH
TAB_F32 = (LENGTH + 1) * EMB
NFULL = LENGTH // 16
TAIL = LENGTH - NFULL * 16


@functools.partial(
    pl.kernel,
    mesh=plsc.VectorSubcoreMesh(core_axis_name="c", subcore_axis_name="s"),
    out_type=jax.ShapeDtypeStruct((BATCH * LENGTH * EMB,), jnp.float32),
    compiler_params=pltpu.CompilerParams(
        needs_layout_passes=False, use_tc_tiling_on_sc=False),
    scratch_types=[
        pltpu.VMEM((XWORDS + 16,), jnp.int32),
        pltpu.VMEM((TAB_F32,), jnp.float32),
        pltpu.VMEM((NBUF, ROW_F32), jnp.float32),
        [pltpu.SemaphoreType.DMA] * NBUF,
    ],
)
def _pos_embed_sc(x_hbm, table_hbm, out_hbm, xv, tab, buf, wsem):
    wid = lax.axis_index("s") * NUM_CORES + lax.axis_index("c")
    tok_base = wid * ROWS_PER_W * LENGTH
    out_base = wid * ROWS_PER_W * ROW_F32

    pltpu.sync_copy(table_hbm, tab)

    lane = lax.iota(jnp.int32, 16)
    ones = jnp.ones((16,), jnp.int32)
    zeros = jnp.zeros((16,), jnp.int32)
    tail_lanes = lane < TAIL

    def stage_x(r):
        pltpu.sync_copy(
            x_hbm.at[pl.ds(tok_base + (r >> 6) * XWORDS, XWORDS)],
            xv.at[pl.ds(0, XWORDS)])

    def chunk_vec(lr, i):
        return xv[pl.ds(lr * LENGTH + 16 * i, 16)]

    def count_zeros(lr):
        def cz(i, n):
            v = chunk_vec(lr, i)
            return n + jnp.sum(jnp.where(v == 0, ones, zeros))
        n = lax.fori_loop(0, NFULL, cz, jnp.int32(0))
        v = chunk_vec(lr, NFULL)
        return n + jnp.sum(jnp.where(tail_lanes & (v == 0), ones, zeros))

    def copy_words(dst, src_word, dst_word, nwords):
        def cw(q, _):
            dst[pl.ds(dst_word + q * 16, 16)] = tab[pl.ds(src_word + q * 16, 16)]
            return _
        lax.fori_loop(0, nwords // 16, cw, jnp.int32(0))

    def token_copies(dst, v, base_tok, ntok, carry):
        c = carry
        for t in range(ntok):
            m = jnp.where(v[t] != 0, jnp.int32(1), jnp.int32(0))
            c = c + m
            src = c * m * EMB
            dw = (base_tok + t) * EMB
            for q in range(0, EMB, 16):
                dst[pl.ds(dw + q, 16)] = tab[pl.ds(src + q, 16)]

    def assemble_row(lr, b):
        dst = buf.at[b]

        def step(i, carry):
            v = chunk_vec(lr, i)
            cnt = jnp.sum(jnp.where(v == 0, ones, zeros))

            @pl.when(cnt == 0)
            def _():
                copy_words(dst, (carry + 1) * EMB, i * (16 * EMB), 16 * EMB)

            @pl.when(cnt > 0)
            def _():
                token_copies(dst, v, i * 16, 16, carry)

            return carry + jnp.int32(16) - cnt

        carry = lax.fori_loop(0, NFULL, step, jnp.int32(0))

        v = chunk_vec(lr, NFULL)
        cnt = jnp.sum(jnp.where(tail_lanes & (v == 0), ones, zeros))

        @pl.when(cnt == 0)
        def _():
            copy_words(dst, (carry + 1) * EMB, NFULL * (16 * EMB),
                       TAIL * EMB)

        @pl.when(cnt > 0)
        def _():
            token_copies(dst, v, NFULL * 16, TAIL, carry)

    def start_write(src, r, b):
        pltpu.async_copy(src,
                         out_hbm.at[pl.ds(out_base + r * ROW_F32, ROW_F32)],
                         wsem[b])

    def wait_write(r, b):
        pltpu.make_async_copy(
            buf.at[b],
            out_hbm.at[pl.ds(out_base + r * ROW_F32, ROW_F32)],
            wsem[b]).wait()

    def body(p, carry):
        for b in range(NBUF):
            r = p * NBUF + b
            lr = r & (XCHUNK - 1)

            @pl.when((r & (XCHUNK - 1)) == 0)
            def _(r=r):
                stage_x(r)

            @pl.when(r >= NBUF)
            def _(r=r, b=b):
                wait_write(r - NBUF, b)

            nzero = count_zeros(lr)

            @pl.when(nzero == 0)
            def _(r=r, b=b):
                start_write(tab.at[pl.ds(EMB, ROW_F32)], r, b)

            @pl.when(nzero > 0)
            def _(lr=lr, r=r, b=b):
                assemble_row(lr, b)
                start_write(buf.at[b], r, b)

        return carry

    lax.fori_loop(0, ROWS_PER_W // NBUF, body, jnp.int32(0))

    for b in range(NBUF):
        wait_write(ROWS_PER_W - NBUF + b, b)


def kernel(X, pos_table):
    out = _pos_embed_sc(X.reshape(BATCH * LENGTH),
                        pos_table.reshape(TAB_F32))
    return out.reshape(BATCH, LENGTH, EMB)


# speedup vs baseline: 3.5196x; 1.0187x over previous
import functools

import jax
import jax.numpy as jnp
from jax import lax
from jax.experimental import pallas as pl
from jax.experimental.pallas import tpu as pltpu
from jax.experimental.pallas import tpu_sc as plsc

LENGTH = 200
EMB = 64
BATCH = 4096

NUM_CORES = 2
NUM_SUBCORES = 16
NW = NUM_CORES * NUM_SUBCORES
ROWS_PER_W = BATCH // NW
ROW_F32 = LENGTH * EMB
NBUF = 8
XCHUNK = 64
XWORDS = XCHUNK * LENGTH
TAB_F32 = (LENGTH + 1) * EMB
NFULL = LENGTH // 16
TAIL = LENGTH - NFULL * 16


@functools.partial(
    pl.kernel,
    mesh=plsc.VectorSubcoreMesh(core_axis_name="c", subcore_axis_name="s"),
    out_type=jax.ShapeDtypeStruct((BATCH * LENGTH * EMB,), jnp.float32),
    compiler_params=pltpu.CompilerParams(
        needs_layout_passes=False, use_tc_tiling_on_sc=False),
    scratch_types=[
        pltpu.VMEM((XWORDS + 16,), jnp.int32),
        pltpu.VMEM((TAB_F32,), jnp.float32),
        pltpu.VMEM((NBUF, ROW_F32), jnp.float32),
        [pltpu.SemaphoreType.DMA] * NBUF,
    ],
)
def _pos_embed_sc(x_hbm, table_hbm, out_hbm, xv, tab, buf, wsem):
    wid = lax.axis_index("s") * NUM_CORES + lax.axis_index("c")
    tok_base = wid * ROWS_PER_W * LENGTH
    out_base = wid * ROWS_PER_W * ROW_F32

    pltpu.sync_copy(table_hbm, tab)

    lane = lax.iota(jnp.int32, 16)
    ones = jnp.ones((16,), jnp.int32)
    zeros = jnp.zeros((16,), jnp.int32)
    tail_lanes = lane < TAIL

    def stage_x(r):
        pltpu.sync_copy(
            x_hbm.at[pl.ds(tok_base + (r >> 6) * XWORDS, XWORDS)],
            xv.at[pl.ds(0, XWORDS)])

    def chunk_vec(lr, i):
        return xv[pl.ds(lr * LENGTH + 16 * i, 16)]

    def count_zeros(lr):
        def cz(i, n):
            v = chunk_vec(lr, i)
            return n + jnp.sum(jnp.where(v == 0, ones, zeros))
        n = lax.fori_loop(0, NFULL, cz, jnp.int32(0))
        v = chunk_vec(lr, NFULL)
        return n + jnp.sum(jnp.where(tail_lanes & (v == 0), ones, zeros))

    def copy_words(dst, src_word, dst_word, nwords):
        def cw(q, _):
            dst[pl.ds(dst_word + q * 16, 16)] = tab[pl.ds(src_word + q * 16, 16)]
            return _
        lax.fori_loop(0, nwords // 16, cw, jnp.int32(0))

    def token_copies(dst, v, base_tok, ntok, carry):
        c = carry
        for t in range(ntok):
            m = jnp.where(v[t] != 0, jnp.int32(1), jnp.int32(0))
            c = c + m
            src = c * m * EMB
            dw = (base_tok + t) * EMB
            for q in range(0, EMB, 16):
                dst[pl.ds(dw + q, 16)] = tab[pl.ds(src + q, 16)]

    def assemble_row(lr, b):
        dst = buf.at[b]

        def step(i, carry):
            v = chunk_vec(lr, i)
            cnt = jnp.sum(jnp.where(v == 0, ones, zeros))

            @pl.when(cnt == 0)
            def _():
                copy_words(dst, (carry + 1) * EMB, i * (16 * EMB), 16 * EMB)

            @pl.when(cnt > 0)
            def _():
                token_copies(dst, v, i * 16, 16, carry)

            return carry + jnp.int32(16) - cnt

        carry = lax.fori_loop(0, NFULL, step, jnp.int32(0))

        v = chunk_vec(lr, NFULL)
        cnt = jnp.sum(jnp.where(tail_lanes & (v == 0), ones, zeros))

        @pl.when(cnt == 0)
        def _():
            copy_words(dst, (carry + 1) * EMB, NFULL * (16 * EMB),
                       TAIL * EMB)

        @pl.when(cnt > 0)
        def _():
            token_copies(dst, v, NFULL * 16, TAIL, carry)

    def start_write(src, r, b):
        pltpu.async_copy(src,
                         out_hbm.at[pl.ds(out_base + r * ROW_F32, ROW_F32)],
                         wsem[b])

    def wait_write(r, b):
        pltpu.make_async_copy(
            buf.at[b],
            out_hbm.at[pl.ds(out_base + r * ROW_F32, ROW_F32)],
            wsem[b]).wait()

    def body(p, carry):
        for b in range(NBUF):
            r = p * NBUF + b
            lr = r & (XCHUNK - 1)

            @pl.when((r & (XCHUNK - 1)) == 0)
            def _(r=r):
                stage_x(r)

            @pl.when(r >= NBUF)
            def _(r=r, b=b):
                wait_write(r - NBUF, b)

            nzero = count_zeros(lr)

            @pl.when(nzero == 0)
            def _(r=r, b=b):
                start_write(tab.at[pl.ds(EMB, ROW_F32)], r, b)

            @pl.when(nzero > 0)
            def _(lr=lr, r=r, b=b):
                assemble_row(lr, b)
                start_write(buf.at[b], r, b)

        return carry

    lax.fori_loop(0, ROWS_PER_W // NBUF, body, jnp.int32(0))

    for b in range(NBUF):
        wait_write(ROWS_PER_W - NBUF + b, b)


def kernel(X, pos_table):
    out = _pos_embed_sc(X.reshape(BATCH * LENGTH),
                        pos_table.reshape(TAB_F32))
    return out.reshape(BATCH, LENGTH, EMB)
